# Initial kernel scaffold; baseline (speedup 1.0000x reference)
import jax
import jax.numpy as jnp
from jax.experimental import pallas as pl


def kernel(features, edge_index, W, b):
    raise NotImplementedError("write your pallas kernel here")


# speedup vs baseline: 3.3424x; 3.3424x over previous
import functools

import jax
import jax.numpy as jnp
from jax import lax
from jax.experimental import pallas as pl
from jax.experimental.pallas import tpu as pltpu
from jax.experimental.pallas import tpu_sc as plsc

N = 10000
E = 320000
D = 128
OUT = 128

NUM_CORES = 2
NUM_SUBCORES = 16
NUM_WORKERS = NUM_CORES * NUM_SUBCORES
GROUP = 128
GROUPS_PER_WORKER = 80
IDX_CHUNK = 4
OUTER = GROUPS_PER_WORKER // IDX_CHUNK
E_PAD = NUM_WORKERS * GROUPS_PER_WORKER * GROUP
ACC_ROWS = 10240
DEG_W = 16
CHUNKS_PER_SUB = ACC_ROWS // GROUP // NUM_SUBCORES


def _sc_aggregate(features, src3, dst3, zf, zd, ones, zidx):
    mesh = plsc.VectorSubcoreMesh(core_axis_name="c", subcore_axis_name="s")

    @functools.partial(
        pl.kernel,
        out_type=[
            jax.ShapeDtypeStruct((NUM_CORES * ACC_ROWS, D), jnp.float32),
            jax.ShapeDtypeStruct((NUM_CORES * ACC_ROWS, DEG_W), jnp.float32),
        ],
        mesh=mesh,
        scratch_types=[
            pltpu.VMEM((IDX_CHUNK, GROUP), jnp.int32),
            pltpu.VMEM((IDX_CHUNK, GROUP), jnp.int32),
            pltpu.VMEM((CHUNKS_PER_SUB, GROUP), jnp.int32),
            pltpu.VMEM((GROUP, D), jnp.float32),
            pltpu.VMEM((GROUP, DEG_W), jnp.float32),
            pltpu.VMEM_SHARED((ACC_ROWS, D), jnp.float32),
            pltpu.VMEM_SHARED((ACC_ROWS, DEG_W), jnp.float32),
            pltpu.SemaphoreType.DMA,
        ],
        compiler_params=pltpu.CompilerParams(use_tc_tiling_on_sc=False),
    )
    def kern(feat_hbm, src_hbm, dst_hbm, zf_hbm, zd_hbm, ones_hbm, zidx_hbm,
             sums_hbm, degs_hbm,
             src_v, dst_v, zi_v, rows_v, small_v, acc_sh, deg_sh, sem):
        c = lax.axis_index("c")
        s = lax.axis_index("s")
        wid = c * NUM_SUBCORES + s

        pltpu.sync_copy(zf_hbm, rows_v)
        pltpu.sync_copy(zd_hbm, small_v)
        pltpu.sync_copy(zidx_hbm.at[s], zi_v)
        for k in range(CHUNKS_PER_SUB):
            pltpu.sync_copy(rows_v, acc_sh.at[zi_v.at[k]])
            pltpu.sync_copy(small_v, deg_sh.at[zi_v.at[k]])
        pltpu.sync_copy(ones_hbm, small_v)
        plsc.subcore_barrier()

        def body(g, carry):
            ibase = wid * GROUPS_PER_WORKER + g * IDX_CHUNK
            pltpu.sync_copy(src_hbm.at[pl.ds(ibase, IDX_CHUNK)], src_v)
            pltpu.sync_copy(dst_hbm.at[pl.ds(ibase, IDX_CHUNK)], dst_v)
            for j in range(IDX_CHUNK):
                pltpu.async_copy(feat_hbm.at[src_v.at[j]], rows_v, sem).wait()
                pltpu.sync_copy(rows_v, acc_sh.at[dst_v.at[j]], add=True)
                pltpu.sync_copy(small_v, deg_sh.at[dst_v.at[j]], add=True)
            return carry
        lax.fori_loop(0, OUTER, body, 0)
        plsc.subcore_barrier()

        for k in range(CHUNKS_PER_SUB):
            chunk = s + NUM_SUBCORES * k
            off = c * ACC_ROWS + chunk * GROUP
            pltpu.async_copy(acc_sh.at[zi_v.at[k]], rows_v, sem).wait()
            pltpu.sync_copy(rows_v, sums_hbm.at[pl.ds(off, GROUP)])
            pltpu.async_copy(deg_sh.at[zi_v.at[k]], small_v, sem).wait()
            pltpu.sync_copy(small_v, degs_hbm.at[pl.ds(off, GROUP)])

    sums, degs = kern(features, src3, dst3, zf, zd, ones, zidx)
    return (sums.reshape(NUM_CORES, ACC_ROWS, D),
            degs.reshape(NUM_CORES, ACC_ROWS, DEG_W))


def _tc_combine_body(feat_ref, sums_ref, degs_ref, w1t_ref, w2t_ref, b_ref,
                     out_ref):
    p = sums_ref[0] + sums_ref[1]
    d16 = degs_ref[0] + degs_ref[1]
    d = jnp.maximum(d16[:, 0:1], 1.0)
    nbr = p / d
    out_ref[...] = (
        jnp.dot(feat_ref[...], w1t_ref[...], preferred_element_type=jnp.float32)
        + jnp.dot(nbr, w2t_ref[...], preferred_element_type=jnp.float32)
        + b_ref[...]
    )


def _tc_combine(features, sums, degs, w1t, w2t, b2):
    BN = 1000
    grid = (N // BN,)
    return pl.pallas_call(
        _tc_combine_body,
        grid=grid,
        in_specs=[
            pl.BlockSpec((BN, D), lambda i: (i, 0)),
            pl.BlockSpec((NUM_CORES, BN, D), lambda i: (0, i, 0)),
            pl.BlockSpec((NUM_CORES, BN, DEG_W), lambda i: (0, i, 0)),
            pl.BlockSpec((D, OUT), lambda i: (0, 0)),
            pl.BlockSpec((D, OUT), lambda i: (0, 0)),
            pl.BlockSpec((1, OUT), lambda i: (0, 0)),
        ],
        out_specs=pl.BlockSpec((BN, OUT), lambda i: (i, 0)),
        out_shape=jax.ShapeDtypeStruct((N, OUT), jnp.float32),
    )(features, sums, degs, w1t, w2t, b2)


def kernel(features, edge_index, W, b):
    src = edge_index[0]
    dst = edge_index[1]
    pad = E_PAD - E
    src_p = jnp.concatenate([src, jnp.zeros((pad,), jnp.int32)])
    dst_p = jnp.concatenate([dst, jnp.full((pad,), N, jnp.int32)])
    src3 = src_p.reshape(NUM_WORKERS * GROUPS_PER_WORKER, GROUP)
    dst3 = dst_p.reshape(NUM_WORKERS * GROUPS_PER_WORKER, GROUP)
    zf = jnp.zeros((GROUP, D), jnp.float32)
    zd = jnp.zeros((GROUP, DEG_W), jnp.float32)
    ones = jnp.ones((GROUP, DEG_W), jnp.float32)
    cid = jnp.arange(NUM_SUBCORES)[:, None] + NUM_SUBCORES * jnp.arange(CHUNKS_PER_SUB)[None, :]
    zidx = (cid[..., None] * GROUP + jnp.arange(GROUP)).astype(jnp.int32)

    sums, degs = _sc_aggregate(features, src3, dst3, zf, zd, ones, zidx)

    w1t = W[:, :D].T
    w2t = W[:, D:].T
    b2 = b.reshape(1, OUT)
    return _tc_combine(features, sums, degs, w1t, w2t, b2)


# speedup vs baseline: 3.5759x; 1.0698x over previous
import functools

import jax
import jax.numpy as jnp
from jax import lax
from jax.experimental import pallas as pl
from jax.experimental.pallas import tpu as pltpu
from jax.experimental.pallas import tpu_sc as plsc

N = 10000
E = 320000
D = 128
OUT = 128

NUM_CORES = 2
NUM_SUBCORES = 16
NUM_WORKERS = NUM_CORES * NUM_SUBCORES
GROUP = 80
GROUPS_PER_WORKER = 128
IDX_CHUNK = 16
OUTER = GROUPS_PER_WORKER // IDX_CHUNK
E_PAD = NUM_WORKERS * GROUPS_PER_WORKER * GROUP
ACC_ROWS = 10240
DEG_W = 8
CHUNKS_PER_SUB = ACC_ROWS // GROUP // NUM_SUBCORES


def _sc_aggregate(features, src3, dst3, zf, zd, ones, zidx):
    mesh = plsc.VectorSubcoreMesh(core_axis_name="c", subcore_axis_name="s")

    @functools.partial(
        pl.kernel,
        out_type=[
            jax.ShapeDtypeStruct((NUM_CORES * ACC_ROWS, D), jnp.float32),
            jax.ShapeDtypeStruct((NUM_CORES * ACC_ROWS, DEG_W), jnp.float32),
        ],
        mesh=mesh,
        scratch_types=[
            pltpu.VMEM((IDX_CHUNK, GROUP), jnp.int32),
            pltpu.VMEM((IDX_CHUNK, GROUP), jnp.int32),
            pltpu.VMEM((CHUNKS_PER_SUB, GROUP), jnp.int32),
            pltpu.VMEM((GROUP, D), jnp.float32),
            pltpu.VMEM((GROUP, D), jnp.float32),
            pltpu.VMEM((GROUP, DEG_W), jnp.float32),
            pltpu.VMEM_SHARED((ACC_ROWS, D), jnp.float32),
            pltpu.VMEM_SHARED((ACC_ROWS, DEG_W), jnp.float32),
            pltpu.SemaphoreType.DMA,
            pltpu.SemaphoreType.DMA,
            pltpu.SemaphoreType.DMA,
        ],
        compiler_params=pltpu.CompilerParams(use_tc_tiling_on_sc=False),
    )
    def kern(feat_hbm, src_hbm, dst_hbm, zf_hbm, zd_hbm, ones_hbm, zidx_hbm,
             sums_hbm, degs_hbm,
             src_v, dst_v, zi_v, rows0_v, rows1_v, small_v, acc_sh, deg_sh,
             gsem, ssem, dsem):
        c = lax.axis_index("c")
        s = lax.axis_index("s")
        wid = c * NUM_SUBCORES + s
        rows = (rows0_v, rows1_v)

        pltpu.sync_copy(zf_hbm, rows0_v)
        pltpu.sync_copy(zd_hbm, small_v)
        pltpu.sync_copy(zidx_hbm.at[s], zi_v)
        for k in range(CHUNKS_PER_SUB):
            pltpu.sync_copy(rows0_v, acc_sh.at[zi_v.at[k]])
            pltpu.sync_copy(small_v, deg_sh.at[zi_v.at[k]])
        pltpu.sync_copy(ones_hbm, small_v)
        plsc.subcore_barrier()

        def chunk_body(g, carry):
            ibase = wid * GROUPS_PER_WORKER + g * IDX_CHUNK
            pltpu.sync_copy(src_hbm.at[pl.ds(ibase, IDX_CHUNK)], src_v)
            pltpu.sync_copy(dst_hbm.at[pl.ds(ibase, IDX_CHUNK)], dst_v)
            pltpu.async_copy(feat_hbm.at[src_v.at[0]], rows[0], gsem)
            for j in range(IDX_CHUNK):
                b = j % 2
                pltpu.make_async_copy(
                    feat_hbm.at[src_v.at[j]], rows[b], gsem).wait()
                if j >= 1:
                    pltpu.make_async_copy(
                        rows[1 - b], acc_sh.at[dst_v.at[j - 1]], ssem).wait()
                if j + 1 < IDX_CHUNK:
                    pltpu.async_copy(
                        feat_hbm.at[src_v.at[j + 1]], rows[1 - b], gsem)
                if j >= 1:
                    pltpu.make_async_copy(
                        small_v, deg_sh.at[dst_v.at[j - 1]], dsem).wait()
                pltpu.async_copy(rows[b], acc_sh.at[dst_v.at[j]], ssem,
                                 add=True)
                pltpu.async_copy(small_v, deg_sh.at[dst_v.at[j]], dsem,
                                 add=True)
            last = IDX_CHUNK - 1
            pltpu.make_async_copy(
                rows[last % 2], acc_sh.at[dst_v.at[last]], ssem).wait()
            pltpu.make_async_copy(
                small_v, deg_sh.at[dst_v.at[last]], dsem).wait()
            return carry
        lax.fori_loop(0, OUTER, chunk_body, 0)
        plsc.subcore_barrier()

        for k in range(CHUNKS_PER_SUB):
            chunk = s + NUM_SUBCORES * k
            off = c * ACC_ROWS + chunk * GROUP
            pltpu.async_copy(acc_sh.at[zi_v.at[k]], rows0_v, gsem).wait()
            pltpu.sync_copy(rows0_v, sums_hbm.at[pl.ds(off, GROUP)])
            pltpu.async_copy(deg_sh.at[zi_v.at[k]], small_v, gsem).wait()
            pltpu.sync_copy(small_v, degs_hbm.at[pl.ds(off, GROUP)])

    sums, degs = kern(features, src3, dst3, zf, zd, ones, zidx)
    return (sums.reshape(NUM_CORES, ACC_ROWS, D),
            degs.reshape(NUM_CORES, ACC_ROWS, DEG_W))


def _tc_combine_body(feat_ref, sums_ref, degs_ref, w1t_ref, w2t_ref, b_ref,
                     out_ref):
    p = sums_ref[0] + sums_ref[1]
    d16 = degs_ref[0] + degs_ref[1]
    d = jnp.maximum(d16[:, 0:1], 1.0)
    nbr = p / d
    out_ref[...] = (
        jnp.dot(feat_ref[...], w1t_ref[...], preferred_element_type=jnp.float32)
        + jnp.dot(nbr, w2t_ref[...], preferred_element_type=jnp.float32)
        + b_ref[...]
    )


def _tc_combine(features, sums, degs, w1t, w2t, b2):
    BN = 1000
    grid = (N // BN,)
    return pl.pallas_call(
        _tc_combine_body,
        grid=grid,
        in_specs=[
            pl.BlockSpec((BN, D), lambda i: (i, 0)),
            pl.BlockSpec((NUM_CORES, BN, D), lambda i: (0, i, 0)),
            pl.BlockSpec((NUM_CORES, BN, DEG_W), lambda i: (0, i, 0)),
            pl.BlockSpec((D, OUT), lambda i: (0, 0)),
            pl.BlockSpec((D, OUT), lambda i: (0, 0)),
            pl.BlockSpec((1, OUT), lambda i: (0, 0)),
        ],
        out_specs=pl.BlockSpec((BN, OUT), lambda i: (i, 0)),
        out_shape=jax.ShapeDtypeStruct((N, OUT), jnp.float32),
    )(features, sums, degs, w1t, w2t, b2)


def kernel(features, edge_index, W, b):
    src = edge_index[0]
    dst = edge_index[1]
    pad = E_PAD - E
    src_p = jnp.concatenate([src, jnp.zeros((pad,), jnp.int32)])
    dst_p = jnp.concatenate([dst, jnp.full((pad,), N, jnp.int32)])
    src3 = src_p.reshape(NUM_WORKERS * GROUPS_PER_WORKER, GROUP)
    dst3 = dst_p.reshape(NUM_WORKERS * GROUPS_PER_WORKER, GROUP)
    zf = jnp.zeros((GROUP, D), jnp.float32)
    zd = jnp.zeros((GROUP, DEG_W), jnp.float32)
    ones = jnp.ones((GROUP, DEG_W), jnp.float32)
    cid = (jnp.arange(NUM_SUBCORES)[:, None]
           + NUM_SUBCORES * jnp.arange(CHUNKS_PER_SUB)[None, :])
    zidx = (cid[..., None] * GROUP + jnp.arange(GROUP)).astype(jnp.int32)

    sums, degs = _sc_aggregate(features, src3, dst3, zf, zd, ones, zidx)

    w1t = W[:, :D].T
    w2t = W[:, D:].T
    b2 = b.reshape(1, OUT)
    return _tc_combine(features, sums, degs, w1t, w2t, b2)


# speedup vs baseline: 9.9120x; 2.7719x over previous
import functools

import jax
import jax.numpy as jnp
from jax import lax
from jax.experimental import pallas as pl
from jax.experimental.pallas import tpu as pltpu
from jax.experimental.pallas import tpu_sc as plsc

N = 10000
E = 320000
D = 128
OUT = 128

NUM_CORES = 2
NUM_SUBCORES = 16
NUM_WORKERS = NUM_CORES * NUM_SUBCORES
GROUP = 80
GROUPS_PER_WORKER = 128
IDX_CHUNK = 16
OUTER = GROUPS_PER_WORKER // IDX_CHUNK
E_PAD = NUM_WORKERS * GROUPS_PER_WORKER * GROUP
ACC_ROWS = 10240
DEG_W = 8
CHUNKS_PER_SUB = ACC_ROWS // GROUP // NUM_SUBCORES


def _sc_aggregate(features, src3, dst3, zf, zd, ones, zidx):
    mesh = plsc.VectorSubcoreMesh(core_axis_name="c", subcore_axis_name="s")

    @functools.partial(
        pl.kernel,
        out_type=[
            jax.ShapeDtypeStruct((NUM_CORES * ACC_ROWS, D), jnp.float32),
            jax.ShapeDtypeStruct((NUM_CORES * ACC_ROWS, DEG_W), jnp.float32),
        ],
        mesh=mesh,
        scratch_types=[
            pltpu.VMEM((IDX_CHUNK, GROUP), jnp.int32),
            pltpu.VMEM((IDX_CHUNK, GROUP), jnp.int32),
            pltpu.VMEM((CHUNKS_PER_SUB, GROUP), jnp.int32),
            pltpu.VMEM((GROUP, D), jnp.float32),
            pltpu.VMEM((GROUP, D), jnp.float32),
            pltpu.VMEM((GROUP, DEG_W), jnp.float32),
            pltpu.VMEM_SHARED((ACC_ROWS, D), jnp.float32),
            pltpu.VMEM_SHARED((ACC_ROWS, DEG_W), jnp.float32),
            pltpu.SemaphoreType.DMA,
            pltpu.SemaphoreType.DMA,
            pltpu.SemaphoreType.DMA,
        ],
        compiler_params=pltpu.CompilerParams(use_tc_tiling_on_sc=False),
    )
    def kern(feat_hbm, src_hbm, dst_hbm, zf_hbm, zd_hbm, ones_hbm, zidx_hbm,
             sums_hbm, degs_hbm,
             src_v, dst_v, zi_v, rows0_v, rows1_v, small_v, acc_sh, deg_sh,
             gsem, ssem, dsem):
        c = lax.axis_index("c")
        s = lax.axis_index("s")
        wid = c * NUM_SUBCORES + s
        rows = (rows0_v, rows1_v)

        pltpu.sync_copy(zf_hbm, rows0_v)
        pltpu.sync_copy(zd_hbm, small_v)
        pltpu.sync_copy(zidx_hbm.at[s], zi_v)
        for k in range(CHUNKS_PER_SUB):
            pltpu.sync_copy(rows0_v, acc_sh.at[zi_v.at[k]])
            pltpu.sync_copy(small_v, deg_sh.at[zi_v.at[k]])
        pltpu.sync_copy(ones_hbm, small_v)
        plsc.subcore_barrier()

        def chunk_body(g, carry):
            ibase = wid * GROUPS_PER_WORKER + g * IDX_CHUNK
            pltpu.sync_copy(src_hbm.at[pl.ds(ibase, IDX_CHUNK)], src_v)
            pltpu.sync_copy(dst_hbm.at[pl.ds(ibase, IDX_CHUNK)], dst_v)
            pltpu.async_copy(feat_hbm.at[src_v.at[0]], rows[0], gsem)
            for j in range(IDX_CHUNK):
                b = j % 2
                pltpu.make_async_copy(
                    feat_hbm.at[src_v.at[j]], rows[b], gsem).wait()
                if j >= 1:
                    pltpu.make_async_copy(
                        rows[1 - b], acc_sh.at[dst_v.at[j - 1]], ssem).wait()
                if j + 1 < IDX_CHUNK:
                    pltpu.async_copy(
                        feat_hbm.at[src_v.at[j + 1]], rows[1 - b], gsem)
                if j >= 1:
                    pltpu.make_async_copy(
                        small_v, deg_sh.at[dst_v.at[j - 1]], dsem).wait()
                pltpu.async_copy(rows[b], acc_sh.at[dst_v.at[j]], ssem,
                                 add=True)
                pltpu.async_copy(small_v, deg_sh.at[dst_v.at[j]], dsem,
                                 add=True)
            last = IDX_CHUNK - 1
            pltpu.make_async_copy(
                rows[last % 2], acc_sh.at[dst_v.at[last]], ssem).wait()
            pltpu.make_async_copy(
                small_v, deg_sh.at[dst_v.at[last]], dsem).wait()
            return carry
        lax.fori_loop(0, OUTER, chunk_body, 0)
        plsc.subcore_barrier()

        for k in range(CHUNKS_PER_SUB):
            chunk = s + NUM_SUBCORES * k
            off = c * ACC_ROWS + chunk * GROUP
            pltpu.async_copy(acc_sh.at[zi_v.at[k]], rows0_v, gsem).wait()
            pltpu.sync_copy(rows0_v, sums_hbm.at[pl.ds(off, GROUP)])
            pltpu.async_copy(deg_sh.at[zi_v.at[k]], small_v, gsem).wait()
            pltpu.sync_copy(small_v, degs_hbm.at[pl.ds(off, GROUP)])

    sums, degs = kern(features, src3, dst3, zf, zd, ones, zidx)
    return (sums.reshape(NUM_CORES, ACC_ROWS, D),
            degs.reshape(NUM_CORES, ACC_ROWS, DEG_W))


def _tc_combine_body(feat_ref, sums_ref, degs_ref, w1t_ref, w2t_ref, b_ref,
                     out_ref):
    p = sums_ref[0] + sums_ref[1]
    d16 = degs_ref[0] + degs_ref[1]
    d = jnp.maximum(d16[:, 0:1], 1.0)
    nbr = p / d
    out_ref[...] = (
        jnp.dot(feat_ref[...], w1t_ref[...], preferred_element_type=jnp.float32)
        + jnp.dot(nbr, w2t_ref[...], preferred_element_type=jnp.float32)
        + b_ref[...]
    )


def _tc_combine(features, sums, degs, w1t, w2t, b2):
    BN = 1000
    grid = (N // BN,)
    return pl.pallas_call(
        _tc_combine_body,
        grid=grid,
        in_specs=[
            pl.BlockSpec((BN, D), lambda i: (i, 0)),
            pl.BlockSpec((NUM_CORES, BN, D), lambda i: (0, i, 0)),
            pl.BlockSpec((NUM_CORES, BN, DEG_W), lambda i: (0, i, 0)),
            pl.BlockSpec((D, OUT), lambda i: (0, 0)),
            pl.BlockSpec((D, OUT), lambda i: (0, 0)),
            pl.BlockSpec((1, OUT), lambda i: (0, 0)),
        ],
        out_specs=pl.BlockSpec((BN, OUT), lambda i: (i, 0)),
        out_shape=jax.ShapeDtypeStruct((N, OUT), jnp.float32),
    )(features, sums, degs, w1t, w2t, b2)


def kernel(features, edge_index, W, b):
    src = edge_index[0]
    dst = edge_index[1]
    pad = E_PAD - E
    filler = jnp.arange(pad, dtype=jnp.int32)
    src_p = jnp.concatenate([src, filler % N])
    dst_p = jnp.concatenate([dst, N + filler % (ACC_ROWS - N)])
    src3 = src_p.reshape(NUM_WORKERS * GROUPS_PER_WORKER, GROUP)
    dst3 = dst_p.reshape(NUM_WORKERS * GROUPS_PER_WORKER, GROUP)
    zf = jnp.zeros((GROUP, D), jnp.float32)
    zd = jnp.zeros((GROUP, DEG_W), jnp.float32)
    ones = jnp.ones((GROUP, DEG_W), jnp.float32)
    cid = (jnp.arange(NUM_SUBCORES)[:, None]
           + NUM_SUBCORES * jnp.arange(CHUNKS_PER_SUB)[None, :])
    zidx = (cid[..., None] * GROUP + jnp.arange(GROUP)).astype(jnp.int32)

    sums, degs = _sc_aggregate(features, src3, dst3, zf, zd, ones, zidx)

    w1t = W[:, :D].T
    w2t = W[:, D:].T
    b2 = b.reshape(1, OUT)
    return _tc_combine(features, sums, degs, w1t, w2t, b2)


# speedup vs baseline: 10.1400x; 1.0230x over previous
import functools

import jax
import jax.numpy as jnp
from jax import lax
from jax.experimental import pallas as pl
from jax.experimental.pallas import tpu as pltpu
from jax.experimental.pallas import tpu_sc as plsc

N = 10000
E = 320000
D = 128
OUT = 128

NUM_CORES = 2
NUM_SUBCORES = 16
NUM_WORKERS = NUM_CORES * NUM_SUBCORES
GROUP = 80
GROUPS_PER_WORKER = 128
IDX_CHUNK = 16
OUTER = GROUPS_PER_WORKER // IDX_CHUNK
E_PAD = NUM_WORKERS * GROUPS_PER_WORKER * GROUP
ACC_ROWS = 10240
DEG_W = 8
CHUNKS_PER_SUB = ACC_ROWS // GROUP // NUM_SUBCORES


def _sc_aggregate(features, src3, dst3, zf, zd, ones, zidx):
    mesh = plsc.VectorSubcoreMesh(core_axis_name="c", subcore_axis_name="s")

    @functools.partial(
        pl.kernel,
        out_type=[
            jax.ShapeDtypeStruct((NUM_CORES * ACC_ROWS, D), jnp.float32),
            jax.ShapeDtypeStruct((NUM_CORES * ACC_ROWS, DEG_W), jnp.float32),
        ],
        mesh=mesh,
        scratch_types=[
            pltpu.VMEM((IDX_CHUNK, GROUP), jnp.int32),
            pltpu.VMEM((IDX_CHUNK, GROUP), jnp.int32),
            pltpu.VMEM((CHUNKS_PER_SUB, GROUP), jnp.int32),
            pltpu.VMEM((GROUP, D), jnp.float32),
            pltpu.VMEM((GROUP, D), jnp.float32),
            pltpu.VMEM((GROUP, DEG_W), jnp.float32),
            pltpu.VMEM_SHARED((ACC_ROWS, D), jnp.float32),
            pltpu.VMEM_SHARED((ACC_ROWS, DEG_W), jnp.float32),
            pltpu.SemaphoreType.DMA,
            pltpu.SemaphoreType.DMA,
            pltpu.SemaphoreType.DMA,
        ],
        compiler_params=pltpu.CompilerParams(use_tc_tiling_on_sc=False),
    )
    def kern(feat_hbm, src_hbm, dst_hbm, zf_hbm, zd_hbm, ones_hbm, zidx_hbm,
             sums_hbm, degs_hbm,
             src_v, dst_v, zi_v, rows0_v, rows1_v, small_v, acc_sh, deg_sh,
             gsem, ssem, dsem):
        c = lax.axis_index("c")
        s = lax.axis_index("s")
        wid = c * NUM_SUBCORES + s
        rows = (rows0_v, rows1_v)

        pltpu.sync_copy(zf_hbm, rows0_v)
        pltpu.sync_copy(zd_hbm, small_v)
        pltpu.sync_copy(zidx_hbm.at[s], zi_v)
        for k in range(CHUNKS_PER_SUB):
            pltpu.async_copy(rows0_v, acc_sh.at[zi_v.at[k]], ssem)
            pltpu.async_copy(small_v, deg_sh.at[zi_v.at[k]], dsem)
        for k in range(CHUNKS_PER_SUB):
            pltpu.make_async_copy(rows0_v, acc_sh.at[zi_v.at[k]], ssem).wait()
            pltpu.make_async_copy(small_v, deg_sh.at[zi_v.at[k]], dsem).wait()
        pltpu.sync_copy(ones_hbm, small_v)
        plsc.subcore_barrier()

        def chunk_body(g, carry):
            ibase = wid * GROUPS_PER_WORKER + g * IDX_CHUNK
            pltpu.sync_copy(src_hbm.at[pl.ds(ibase, IDX_CHUNK)], src_v)
            pltpu.sync_copy(dst_hbm.at[pl.ds(ibase, IDX_CHUNK)], dst_v)
            pltpu.async_copy(feat_hbm.at[src_v.at[0]], rows[0], gsem)
            for j in range(IDX_CHUNK):
                b = j % 2
                pltpu.make_async_copy(
                    feat_hbm.at[src_v.at[j]], rows[b], gsem).wait()
                if j >= 1:
                    pltpu.make_async_copy(
                        rows[1 - b], acc_sh.at[dst_v.at[j - 1]], ssem).wait()
                if j + 1 < IDX_CHUNK:
                    pltpu.async_copy(
                        feat_hbm.at[src_v.at[j + 1]], rows[1 - b], gsem)
                if j >= 1:
                    pltpu.make_async_copy(
                        small_v, deg_sh.at[dst_v.at[j - 1]], dsem).wait()
                pltpu.async_copy(rows[b], acc_sh.at[dst_v.at[j]], ssem,
                                 add=True)
                pltpu.async_copy(small_v, deg_sh.at[dst_v.at[j]], dsem,
                                 add=True)
            last = IDX_CHUNK - 1
            pltpu.make_async_copy(
                rows[last % 2], acc_sh.at[dst_v.at[last]], ssem).wait()
            pltpu.make_async_copy(
                small_v, deg_sh.at[dst_v.at[last]], dsem).wait()
            return carry
        lax.fori_loop(0, OUTER, chunk_body, 0)
        plsc.subcore_barrier()

        def _off(k):
            chunk = s + NUM_SUBCORES * k
            return c * ACC_ROWS + chunk * GROUP

        pltpu.async_copy(acc_sh.at[zi_v.at[0]], rows0_v, gsem)
        for k in range(CHUNKS_PER_SUB):
            b = k % 2
            pltpu.make_async_copy(acc_sh.at[zi_v.at[k]], rows[b], gsem).wait()
            if k >= 1:
                pltpu.make_async_copy(
                    rows[1 - b], sums_hbm.at[pl.ds(_off(k - 1), GROUP)],
                    ssem).wait()
            if k + 1 < CHUNKS_PER_SUB:
                pltpu.async_copy(acc_sh.at[zi_v.at[k + 1]], rows[1 - b], gsem)
            pltpu.async_copy(rows[b], sums_hbm.at[pl.ds(_off(k), GROUP)], ssem)
            pltpu.async_copy(deg_sh.at[zi_v.at[k]], small_v, dsem).wait()
            pltpu.sync_copy(small_v, degs_hbm.at[pl.ds(_off(k), GROUP)])
        last = CHUNKS_PER_SUB - 1
        pltpu.make_async_copy(
            rows[last % 2], sums_hbm.at[pl.ds(_off(last), GROUP)], ssem).wait()

    sums, degs = kern(features, src3, dst3, zf, zd, ones, zidx)
    return (sums.reshape(NUM_CORES, ACC_ROWS, D),
            degs.reshape(NUM_CORES, ACC_ROWS, DEG_W))


def _tc_combine_body(feat_ref, sums_ref, degs_ref, w1t_ref, w2t_ref, b_ref,
                     out_ref):
    p = sums_ref[0] + sums_ref[1]
    d16 = degs_ref[0] + degs_ref[1]
    d = jnp.maximum(d16[:, 0:1], 1.0)
    nbr = p / d
    out_ref[...] = (
        jnp.dot(feat_ref[...], w1t_ref[...], preferred_element_type=jnp.float32)
        + jnp.dot(nbr, w2t_ref[...], preferred_element_type=jnp.float32)
        + b_ref[...]
    )


def _tc_combine(features, sums, degs, w1t, w2t, b2):
    BN = 1000
    grid = (N // BN,)
    return pl.pallas_call(
        _tc_combine_body,
        grid=grid,
        in_specs=[
            pl.BlockSpec((BN, D), lambda i: (i, 0)),
            pl.BlockSpec((NUM_CORES, BN, D), lambda i: (0, i, 0)),
            pl.BlockSpec((NUM_CORES, BN, DEG_W), lambda i: (0, i, 0)),
            pl.BlockSpec((D, OUT), lambda i: (0, 0)),
            pl.BlockSpec((D, OUT), lambda i: (0, 0)),
            pl.BlockSpec((1, OUT), lambda i: (0, 0)),
        ],
        out_specs=pl.BlockSpec((BN, OUT), lambda i: (i, 0)),
        out_shape=jax.ShapeDtypeStruct((N, OUT), jnp.float32),
    )(features, sums, degs, w1t, w2t, b2)


def kernel(features, edge_index, W, b):
    src = edge_index[0]
    dst = edge_index[1]
    pad = E_PAD - E
    filler = jnp.arange(pad, dtype=jnp.int32)
    src_p = jnp.concatenate([src, filler % N])
    dst_p = jnp.concatenate([dst, N + filler % (ACC_ROWS - N)])
    src3 = src_p.reshape(NUM_WORKERS * GROUPS_PER_WORKER, GROUP)
    dst3 = dst_p.reshape(NUM_WORKERS * GROUPS_PER_WORKER, GROUP)
    zf = jnp.zeros((GROUP, D), jnp.float32)
    zd = jnp.zeros((GROUP, DEG_W), jnp.float32)
    ones = jnp.ones((GROUP, DEG_W), jnp.float32)
    cid = (jnp.arange(NUM_SUBCORES)[:, None]
           + NUM_SUBCORES * jnp.arange(CHUNKS_PER_SUB)[None, :])
    zidx = (cid[..., None] * GROUP + jnp.arange(GROUP)).astype(jnp.int32)

    sums, degs = _sc_aggregate(features, src3, dst3, zf, zd, ones, zidx)

    w1t = W[:, :D].T
    w2t = W[:, D:].T
    b2 = b.reshape(1, OUT)
    return _tc_combine(features, sums, degs, w1t, w2t, b2)
